# baseline (speedup 1.0000x reference)
import jax
import jax.numpy as jnp
from jax.experimental import pallas as pl

_BT = 256
_BETA = 0.25


def _bf16_dot(a, b, dn):
    return jax.lax.dot_general(
        a.astype(jnp.bfloat16), b.astype(jnp.bfloat16), dimension_numbers=dn,
        preferred_element_type=jnp.float32)


def _vq_stage(res, cb, cb_bf):
    k = cb.shape[0]
    res_nrm = jnp.sum(res * res, axis=1, keepdims=True)
    cross = jax.lax.dot_general(
        res.astype(jnp.bfloat16), cb_bf,
        dimension_numbers=(((1,), (1,)), ((), ())),
        preferred_element_type=jnp.float32)
    cb_sq = cb * cb
    cb_nrm = jax.lax.dot_general(
        jnp.ones((1, cb.shape[1]), jnp.float32), cb_sq,
        dimension_numbers=(((1,), (1,)), ((), ())),
        preferred_element_type=jnp.float32,
        precision=jax.lax.Precision.HIGHEST)
    d = res_nrm - 2.0 * cross + cb_nrm
    m = jnp.min(d, axis=1, keepdims=True)
    iota = jax.lax.broadcasted_iota(jnp.int32, d.shape, 1)
    idx = jnp.min(jnp.where(d == m, iota, k), axis=1)
    onehot = (iota == idx[:, None]).astype(jnp.float32)
    zq = jnp.dot(onehot, cb, preferred_element_type=jnp.float32,
                 precision=jax.lax.Precision.HIGHEST)
    diff = res - zq
    loss = (1.0 + _BETA) * jnp.mean(diff * diff, axis=1)
    return zq, loss, idx


def _body(x_ref, eW0, eb0, eW1, eb1, eW2, eb2,
          dW0, db0, dW1, db1, dW2, db2, ln_g, ln_b,
          cb0, cb1, cb2,
          xhat_ref, loss_ref, i0_ref, i1_ref, i2_ref):
    x = x_ref[...]
    mm = (((1,), (0,)), ((), ()))
    h = jnp.maximum(_bf16_dot(x, eW0[...], mm) + eb0[...], 0.0)
    h = jnp.maximum(_bf16_dot(h, eW1[...], mm) + eb1[...], 0.0)
    z = _bf16_dot(h, eW2[...], mm) + eb2[...]

    mu = jnp.mean(z, axis=-1, keepdims=True)
    zc = z - mu
    var = jnp.mean(zc * zc, axis=-1, keepdims=True)
    z = zc / jnp.sqrt(var + 1e-5) * ln_g[...] + ln_b[...]

    c0, c1, c2 = cb0[...], cb1[...], cb2[...]
    zq0, l0, i0 = _vq_stage(z, c0, c0.astype(jnp.bfloat16))
    r1 = z - zq0
    zq1, l1, i1 = _vq_stage(r1, c1, c1.astype(jnp.bfloat16))
    r2 = r1 - zq1
    zq2, l2, i2 = _vq_stage(r2, c2, c2.astype(jnp.bfloat16))

    zqt = zq0 + zq1 + zq2
    h = jnp.maximum(_bf16_dot(zqt, dW0[...], mm) + db0[...], 0.0)
    h = jnp.maximum(_bf16_dot(h, dW1[...], mm) + db1[...], 0.0)
    xh = _bf16_dot(h, dW2[...], mm) + db2[...]

    dx = xh - x
    recon = jnp.mean(dx * dx, axis=1)
    loss = l0 + l1 + l2 + recon

    xhat_ref[...] = xh
    loss_ref[...] = loss[:, None]
    i0_ref[...] = i0[:, None]
    i1_ref[...] = i1[:, None]
    i2_ref[...] = i2[:, None]


def kernel(x, enc_W0, enc_b0, enc_W1, enc_b1, enc_W2, enc_b2,
           dec_W0, dec_b0, dec_W1, dec_b1, dec_W2, dec_b2,
           ln_g, ln_b, cb0, cb1, cb2):
    batch, in_dim = x.shape
    grid = (batch // _BT,)

    def rows(i):
        return (i, 0)

    def whole(i):
        return (0, 0)

    row_spec = lambda n: pl.BlockSpec((_BT, n), rows)
    full_spec = lambda a: pl.BlockSpec(a.shape, whole)

    b2 = lambda v: v.reshape(1, -1)
    ins = (x,
           enc_W0, b2(enc_b0), enc_W1, b2(enc_b1), enc_W2, b2(enc_b2),
           dec_W0, b2(dec_b0), dec_W1, b2(dec_b1), dec_W2, b2(dec_b2),
           b2(ln_g), b2(ln_b), cb0, cb1, cb2)
    in_specs = [row_spec(in_dim)] + [full_spec(a) for a in ins[1:]]

    out_shape = (
        jax.ShapeDtypeStruct((batch, in_dim), jnp.float32),
        jax.ShapeDtypeStruct((batch, 1), jnp.float32),
        jax.ShapeDtypeStruct((batch, 1), jnp.int32),
        jax.ShapeDtypeStruct((batch, 1), jnp.int32),
        jax.ShapeDtypeStruct((batch, 1), jnp.int32),
    )
    out_specs = (
        row_spec(in_dim),
        pl.BlockSpec((_BT, 1), rows),
        pl.BlockSpec((_BT, 1), rows),
        pl.BlockSpec((_BT, 1), rows),
        pl.BlockSpec((_BT, 1), rows),
    )

    xh, loss, i0, i1, i2 = pl.pallas_call(
        _body,
        grid=grid,
        in_specs=in_specs,
        out_specs=out_specs,
        out_shape=out_shape,
    )(*ins)

    indices = jnp.concatenate([i0, i1, i2], axis=1)
    return xh, loss.reshape(batch), indices


# speedup vs baseline: 3.3251x; 3.3251x over previous
import jax
import jax.numpy as jnp
from jax.experimental import pallas as pl

_BT = 256
_BETA = 0.25

_MM = (((1,), (0,)), ((), ()))
_MMT = (((1,), (1,)), ((), ()))


def _bdot(a, b, dn=_MM):
    return jax.lax.dot_general(a, b, dimension_numbers=dn,
                               preferred_element_type=jnp.float32)


def _vq_stage(res, cb_hi, cb_mid, cb_lo, cb_nrm):
    k = cb_hi.shape[0]
    res_nrm = jnp.sum(res * res, axis=1, keepdims=True)
    cross = _bdot(res.astype(jnp.bfloat16), cb_hi, _MMT)
    d = res_nrm - 2.0 * cross + cb_nrm
    m = jnp.min(d, axis=1, keepdims=True)
    iota = jax.lax.broadcasted_iota(jnp.int32, d.shape, 1)
    idx = jnp.min(jnp.where(d == m, iota, k), axis=1)
    onehot = (iota == idx[:, None]).astype(jnp.bfloat16)
    zq = _bdot(onehot, cb_hi) + _bdot(onehot, cb_mid) + _bdot(onehot, cb_lo)
    diff = res - zq
    loss = (1.0 + _BETA) * jnp.mean(diff * diff, axis=1)
    return zq, loss, idx


def _body(x_ref, eW0, eb0, eW1, eb1, eW2, eb2,
          dW0, db0, dW1, db1, dW2, db2, ln_g, ln_b,
          h0, m0, l0_, n0, h1, m1, l1_, n1, h2, m2, l2_, n2,
          xhat_ref, loss_ref, i0_ref, i1_ref, i2_ref):
    x = x_ref[...]
    xb = x.astype(jnp.bfloat16)
    h = jnp.maximum(_bdot(xb, eW0[...]) + eb0[...], 0.0)
    h = jnp.maximum(_bdot(h.astype(jnp.bfloat16), eW1[...]) + eb1[...], 0.0)
    z = _bdot(h.astype(jnp.bfloat16), eW2[...]) + eb2[...]

    mu = jnp.mean(z, axis=-1, keepdims=True)
    zc = z - mu
    var = jnp.mean(zc * zc, axis=-1, keepdims=True)
    z = zc / jnp.sqrt(var + 1e-5) * ln_g[...] + ln_b[...]

    zq0, l0, i0 = _vq_stage(z, h0[...], m0[...], l0_[...], n0[...])
    r1 = z - zq0
    zq1, l1, i1 = _vq_stage(r1, h1[...], m1[...], l1_[...], n1[...])
    r2 = r1 - zq1
    zq2, l2, i2 = _vq_stage(r2, h2[...], m2[...], l2_[...], n2[...])

    zqt = zq0 + zq1 + zq2
    h = jnp.maximum(_bdot(zqt.astype(jnp.bfloat16), dW0[...]) + db0[...], 0.0)
    h = jnp.maximum(_bdot(h.astype(jnp.bfloat16), dW1[...]) + db1[...], 0.0)
    xh = _bdot(h.astype(jnp.bfloat16), dW2[...]) + db2[...]

    dx = xh - x
    recon = jnp.mean(dx * dx, axis=1)
    loss = l0 + l1 + l2 + recon

    xhat_ref[...] = xh
    loss_ref[...] = loss[:, None]
    i0_ref[...] = i0[:, None]
    i1_ref[...] = i1[:, None]
    i2_ref[...] = i2[:, None]


def _split3(cb):
    hi = cb.astype(jnp.bfloat16)
    r1 = cb - hi.astype(jnp.float32)
    mid = r1.astype(jnp.bfloat16)
    lo = (r1 - mid.astype(jnp.float32)).astype(jnp.bfloat16)
    return hi, mid, lo


def kernel(x, enc_W0, enc_b0, enc_W1, enc_b1, enc_W2, enc_b2,
           dec_W0, dec_b0, dec_W1, dec_b1, dec_W2, dec_b2,
           ln_g, ln_b, cb0, cb1, cb2):
    batch, in_dim = x.shape
    grid = (batch // _BT,)

    def rows(i):
        return (i, 0)

    def whole(i):
        return (0, 0)

    row_spec = lambda n: pl.BlockSpec((_BT, n), rows)
    full_spec = lambda a: pl.BlockSpec(a.shape, whole)

    b2 = lambda v: v.reshape(1, -1)
    bw = lambda w: w.astype(jnp.bfloat16)
    cb_ins = []
    for cb in (cb0, cb1, cb2):
        hi, mid, lo = _split3(cb)
        nrm = jnp.sum(cb * cb, axis=1)[None, :]
        cb_ins += [hi, mid, lo, nrm]

    ins = (x,
           bw(enc_W0), b2(enc_b0), bw(enc_W1), b2(enc_b1), bw(enc_W2), b2(enc_b2),
           bw(dec_W0), b2(dec_b0), bw(dec_W1), b2(dec_b1), bw(dec_W2), b2(dec_b2),
           b2(ln_g), b2(ln_b), *cb_ins)
    in_specs = [row_spec(in_dim)] + [full_spec(a) for a in ins[1:]]

    out_shape = (
        jax.ShapeDtypeStruct((batch, in_dim), jnp.float32),
        jax.ShapeDtypeStruct((batch, 1), jnp.float32),
        jax.ShapeDtypeStruct((batch, 1), jnp.int32),
        jax.ShapeDtypeStruct((batch, 1), jnp.int32),
        jax.ShapeDtypeStruct((batch, 1), jnp.int32),
    )
    out_specs = (
        row_spec(in_dim),
        pl.BlockSpec((_BT, 1), rows),
        pl.BlockSpec((_BT, 1), rows),
        pl.BlockSpec((_BT, 1), rows),
        pl.BlockSpec((_BT, 1), rows),
    )

    xh, loss, i0, i1, i2 = pl.pallas_call(
        _body,
        grid=grid,
        in_specs=in_specs,
        out_specs=out_specs,
        out_shape=out_shape,
    )(*ins)

    indices = jnp.concatenate([i0, i1, i2], axis=1)
    return xh, loss.reshape(batch), indices


# speedup vs baseline: 4.8598x; 1.4615x over previous
import jax
import jax.numpy as jnp
from jax.experimental import pallas as pl

_BT = 256
_BETA = 0.25

_MM = (((1,), (0,)), ((), ()))
_MMT = (((1,), (1,)), ((), ()))


def _bdot(a, b, dn=_MM):
    return jax.lax.dot_general(a, b, dimension_numbers=dn,
                               preferred_element_type=jnp.float32)


def _vq_stage(res, cb_hi, cb_cat, cb_nrm):
    res_nrm = jnp.sum(res * res, axis=1, keepdims=True)
    crossm2 = _bdot((-2.0 * res).astype(jnp.bfloat16), cb_hi, _MMT)
    d = (res_nrm + crossm2) + cb_nrm
    idx = jnp.argmin(d, axis=1).astype(jnp.int32)
    iota = jax.lax.broadcasted_iota(jnp.int32, d.shape, 1)
    onehot = (iota == idx[:, None]).astype(jnp.bfloat16)
    lat = res.shape[1]
    g = _bdot(onehot, cb_cat)
    zq = (g[:, :lat] + g[:, lat:2 * lat]) + g[:, 2 * lat:]
    diff = res - zq
    loss = (1.0 + _BETA) * jnp.mean(diff * diff, axis=1)
    return zq, loss, idx


def _body(x_ref, eW0, eb0, eW1, eb1, eW2, eb2,
          dW0, db0, dW1, db1, dW2, db2, ln_g, ln_b,
          h0, t0, n0, h1, t1, n1, h2, t2, n2,
          xhat_ref, loss_ref, i0_ref, i1_ref, i2_ref):
    x = x_ref[...]
    xb = x.astype(jnp.bfloat16)
    h = jnp.maximum(_bdot(xb, eW0[...]) + eb0[...], 0.0)
    h = jnp.maximum(_bdot(h.astype(jnp.bfloat16), eW1[...]) + eb1[...], 0.0)
    z = _bdot(h.astype(jnp.bfloat16), eW2[...]) + eb2[...]

    mu = jnp.mean(z, axis=-1, keepdims=True)
    zc = z - mu
    var = jnp.mean(zc * zc, axis=-1, keepdims=True)
    z = zc / jnp.sqrt(var + 1e-5) * ln_g[...] + ln_b[...]

    zq0, l0, i0 = _vq_stage(z, h0[...], t0[...], n0[...])
    r1 = z - zq0
    zq1, l1, i1 = _vq_stage(r1, h1[...], t1[...], n1[...])
    r2 = r1 - zq1
    zq2, l2, i2 = _vq_stage(r2, h2[...], t2[...], n2[...])

    zqt = zq0 + zq1 + zq2
    h = jnp.maximum(_bdot(zqt.astype(jnp.bfloat16), dW0[...]) + db0[...], 0.0)
    h = jnp.maximum(_bdot(h.astype(jnp.bfloat16), dW1[...]) + db1[...], 0.0)
    xh = _bdot(h.astype(jnp.bfloat16), dW2[...]) + db2[...]

    dx = xh - x
    recon = jnp.mean(dx * dx, axis=1)
    loss = l0 + l1 + l2 + recon

    xhat_ref[...] = xh
    loss_ref[...] = loss[:, None]
    i0_ref[...] = i0[:, None]
    i1_ref[...] = i1[:, None]
    i2_ref[...] = i2[:, None]


def _split3(cb):
    hi = cb.astype(jnp.bfloat16)
    r1 = cb - hi.astype(jnp.float32)
    mid = r1.astype(jnp.bfloat16)
    lo = (r1 - mid.astype(jnp.float32)).astype(jnp.bfloat16)
    return hi, mid, lo


def kernel(x, enc_W0, enc_b0, enc_W1, enc_b1, enc_W2, enc_b2,
           dec_W0, dec_b0, dec_W1, dec_b1, dec_W2, dec_b2,
           ln_g, ln_b, cb0, cb1, cb2):
    batch, in_dim = x.shape
    grid = (batch // _BT,)

    def rows(i):
        return (i, 0)

    def whole(i):
        return (0, 0)

    row_spec = lambda n: pl.BlockSpec((_BT, n), rows)
    full_spec = lambda a: pl.BlockSpec(a.shape, whole)

    b2 = lambda v: v.reshape(1, -1)
    bw = lambda w: w.astype(jnp.bfloat16)
    cb_ins = []
    for cb in (cb0, cb1, cb2):
        nrm = jnp.sum(cb * cb, axis=1)[None, :]
        hi, mid, lo = _split3(cb)
        cb_ins += [hi, jnp.concatenate([hi, mid, lo], axis=1), nrm]

    ins = (x,
           bw(enc_W0), b2(enc_b0), bw(enc_W1), b2(enc_b1), bw(enc_W2), b2(enc_b2),
           bw(dec_W0), b2(dec_b0), bw(dec_W1), b2(dec_b1), bw(dec_W2), b2(dec_b2),
           b2(ln_g), b2(ln_b), *cb_ins)
    in_specs = [row_spec(in_dim)] + [full_spec(a) for a in ins[1:]]

    out_shape = (
        jax.ShapeDtypeStruct((batch, in_dim), jnp.float32),
        jax.ShapeDtypeStruct((batch, 1), jnp.float32),
        jax.ShapeDtypeStruct((batch, 1), jnp.int32),
        jax.ShapeDtypeStruct((batch, 1), jnp.int32),
        jax.ShapeDtypeStruct((batch, 1), jnp.int32),
    )
    out_specs = (
        row_spec(in_dim),
        pl.BlockSpec((_BT, 1), rows),
        pl.BlockSpec((_BT, 1), rows),
        pl.BlockSpec((_BT, 1), rows),
        pl.BlockSpec((_BT, 1), rows),
    )

    xh, loss, i0, i1, i2 = pl.pallas_call(
        _body,
        grid=grid,
        in_specs=in_specs,
        out_specs=out_specs,
        out_shape=out_shape,
    )(*ins)

    indices = jnp.concatenate([i0, i1, i2], axis=1)
    return xh, loss.reshape(batch), indices


# speedup vs baseline: 5.3163x; 1.0939x over previous
import jax
import jax.numpy as jnp
from jax.experimental import pallas as pl
from jax.experimental.pallas import tpu as pltpu

_BT = 256
_BETA = 0.25

_MM = (((1,), (0,)), ((), ()))
_MMT = (((1,), (1,)), ((), ()))


def _bdot(a, b, dn=_MM):
    return jax.lax.dot_general(a, b, dimension_numbers=dn,
                               preferred_element_type=jnp.float32)


def _vq_stage(res, cb_hi, cb_cat, cb_nrm):
    res_nrm = jnp.sum(res * res, axis=1, keepdims=True)
    crossm2 = _bdot((-2.0 * res).astype(jnp.bfloat16), cb_hi, _MMT)
    d = (res_nrm + crossm2) + cb_nrm
    idx = jnp.argmin(d, axis=1).astype(jnp.int32)
    iota = jax.lax.broadcasted_iota(jnp.int32, d.shape, 1)
    onehot = (iota == idx[:, None]).astype(jnp.bfloat16)
    lat = res.shape[1]
    g = _bdot(onehot, cb_cat)
    zq = (g[:, :lat] + g[:, lat:2 * lat]) + g[:, 2 * lat:]
    diff = res - zq
    loss = (1.0 + _BETA) * jnp.mean(diff * diff, axis=1)
    return zq, loss, idx


def _split_cat(cb):
    hi = cb.astype(jnp.bfloat16)
    r1 = cb - hi.astype(jnp.float32)
    mid = r1.astype(jnp.bfloat16)
    lo = (r1 - mid.astype(jnp.float32)).astype(jnp.bfloat16)
    return hi, jnp.concatenate([hi, mid, lo], axis=1)


def _body(x_ref, eW0, eb0, eW1, eb1, eW2, eb2,
          dW0, db0, dW1, db1, dW2, db2, ln_g, ln_b,
          cb0, cb1, cb2,
          xhat_ref, loss_ref, i0_ref, i1_ref, i2_ref,
          we0, we1, we2, wd0, wd1, wd2,
          hi0, hi1, hi2, cat0, cat1, cat2, nrm0, nrm1, nrm2):

    @pl.when(pl.program_id(0) == 0)
    def _prep():
        we0[...] = eW0[...].astype(jnp.bfloat16)
        we1[...] = eW1[...].astype(jnp.bfloat16)
        we2[...] = eW2[...].astype(jnp.bfloat16)
        wd0[...] = dW0[...].astype(jnp.bfloat16)
        wd1[...] = dW1[...].astype(jnp.bfloat16)
        wd2[...] = dW2[...].astype(jnp.bfloat16)
        for cb_ref, hi_ref, cat_ref, nrm_ref in (
                (cb0, hi0, cat0, nrm0), (cb1, hi1, cat1, nrm1),
                (cb2, hi2, cat2, nrm2)):
            cb = cb_ref[...]
            hi, cat = _split_cat(cb)
            hi_ref[...] = hi
            cat_ref[...] = cat
            nrm_ref[...] = jnp.sum(cb * cb, axis=1)[None, :]

    x = x_ref[...]
    xb = x.astype(jnp.bfloat16)
    h = jnp.maximum(_bdot(xb, we0[...]) + eb0[...], 0.0)
    h = jnp.maximum(_bdot(h.astype(jnp.bfloat16), we1[...]) + eb1[...], 0.0)
    z = _bdot(h.astype(jnp.bfloat16), we2[...]) + eb2[...]

    mu = jnp.mean(z, axis=-1, keepdims=True)
    zc = z - mu
    var = jnp.mean(zc * zc, axis=-1, keepdims=True)
    z = zc / jnp.sqrt(var + 1e-5) * ln_g[...] + ln_b[...]

    zq0, l0, i0 = _vq_stage(z, hi0[...], cat0[...], nrm0[...])
    r1 = z - zq0
    zq1, l1, i1 = _vq_stage(r1, hi1[...], cat1[...], nrm1[...])
    r2 = r1 - zq1
    zq2, l2, i2 = _vq_stage(r2, hi2[...], cat2[...], nrm2[...])

    zqt = zq0 + zq1 + zq2
    h = jnp.maximum(_bdot(zqt.astype(jnp.bfloat16), wd0[...]) + db0[...], 0.0)
    h = jnp.maximum(_bdot(h.astype(jnp.bfloat16), wd1[...]) + db1[...], 0.0)
    xh = _bdot(h.astype(jnp.bfloat16), wd2[...]) + db2[...]

    dx = xh - x
    recon = jnp.mean(dx * dx, axis=1)
    loss = l0 + l1 + l2 + recon

    xhat_ref[...] = xh
    loss_ref[...] = loss[:, None]
    i0_ref[...] = i0[:, None]
    i1_ref[...] = i1[:, None]
    i2_ref[...] = i2[:, None]


def kernel(x, enc_W0, enc_b0, enc_W1, enc_b1, enc_W2, enc_b2,
           dec_W0, dec_b0, dec_W1, dec_b1, dec_W2, dec_b2,
           ln_g, ln_b, cb0, cb1, cb2):
    batch, in_dim = x.shape
    k, lat = cb0.shape
    grid = (batch // _BT,)

    def rows(i):
        return (i, 0)

    def whole(i):
        return (0, 0)

    row_spec = lambda n: pl.BlockSpec((_BT, n), rows)
    full_spec = lambda a: pl.BlockSpec(a.shape, whole)

    b2 = lambda v: v.reshape(1, -1)
    ins = (x,
           enc_W0, b2(enc_b0), enc_W1, b2(enc_b1), enc_W2, b2(enc_b2),
           dec_W0, b2(dec_b0), dec_W1, b2(dec_b1), dec_W2, b2(dec_b2),
           b2(ln_g), b2(ln_b), cb0, cb1, cb2)
    in_specs = [row_spec(in_dim)] + [full_spec(a) for a in ins[1:]]

    out_shape = (
        jax.ShapeDtypeStruct((batch, in_dim), jnp.float32),
        jax.ShapeDtypeStruct((batch, 1), jnp.float32),
        jax.ShapeDtypeStruct((batch, 1), jnp.int32),
        jax.ShapeDtypeStruct((batch, 1), jnp.int32),
        jax.ShapeDtypeStruct((batch, 1), jnp.int32),
    )
    out_specs = (
        row_spec(in_dim),
        pl.BlockSpec((_BT, 1), rows),
        pl.BlockSpec((_BT, 1), rows),
        pl.BlockSpec((_BT, 1), rows),
        pl.BlockSpec((_BT, 1), rows),
    )

    bf = jnp.bfloat16
    scratch = [
        pltpu.VMEM(enc_W0.shape, bf), pltpu.VMEM(enc_W1.shape, bf),
        pltpu.VMEM(enc_W2.shape, bf), pltpu.VMEM(dec_W0.shape, bf),
        pltpu.VMEM(dec_W1.shape, bf), pltpu.VMEM(dec_W2.shape, bf),
        pltpu.VMEM((k, lat), bf), pltpu.VMEM((k, lat), bf),
        pltpu.VMEM((k, lat), bf),
        pltpu.VMEM((k, 3 * lat), bf), pltpu.VMEM((k, 3 * lat), bf),
        pltpu.VMEM((k, 3 * lat), bf),
        pltpu.VMEM((1, k), jnp.float32), pltpu.VMEM((1, k), jnp.float32),
        pltpu.VMEM((1, k), jnp.float32),
    ]

    xh, loss, i0, i1, i2 = pl.pallas_call(
        _body,
        grid=grid,
        in_specs=in_specs,
        out_specs=out_specs,
        out_shape=out_shape,
        scratch_shapes=scratch,
    )(*ins)

    indices = jnp.concatenate([i0, i1, i2], axis=1)
    return xh, loss.reshape(batch), indices


# speedup vs baseline: 5.4509x; 1.0253x over previous
import jax
import jax.numpy as jnp
from jax.experimental import pallas as pl
from jax.experimental.pallas import tpu as pltpu

_BT = 256
_BETA = 0.25

_MM = (((1,), (0,)), ((), ()))
_MMT = (((1,), (1,)), ((), ()))


def _bdot(a, b, dn=_MM):
    return jax.lax.dot_general(a, b, dimension_numbers=dn,
                               preferred_element_type=jnp.float32)


def _vq_stage(res, cb_hi, cb_cat, cb_nrm):
    res_nrm = jnp.sum(res * res, axis=1, keepdims=True)
    crossm2 = _bdot((-2.0 * res).astype(jnp.bfloat16), cb_hi, _MMT)
    d = (res_nrm + crossm2) + cb_nrm
    idx = jnp.argmin(d, axis=1).astype(jnp.int32)
    iota = jax.lax.broadcasted_iota(jnp.int32, d.shape, 1)
    onehot = (iota == idx[:, None]).astype(jnp.bfloat16)
    lat = res.shape[1]
    g = _bdot(onehot, cb_cat)
    zq = (g[:, :lat] + g[:, lat:2 * lat]) + g[:, 2 * lat:]
    diff = res - zq
    loss = (1.0 + _BETA) * jnp.mean(diff * diff, axis=1)
    return zq, loss, idx


def _split_cat(cb):
    hi = cb.astype(jnp.bfloat16)
    r1 = cb - hi.astype(jnp.float32)
    mid = r1.astype(jnp.bfloat16)
    lo = (r1 - mid.astype(jnp.float32)).astype(jnp.bfloat16)
    return hi, jnp.concatenate([hi, mid, lo], axis=1)


def _body(x_ref, eW0, eb0, eW1, eb1, eW2, eb2,
          dW0, db0, dW1, db1, dW2, db2, ln_g, ln_b,
          cb0, cb1, cb2,
          xhat_ref, loss_ref, idx_ref,
          we0, we1, we2, wd0, wd1, wd2,
          hi0, hi1, hi2, cat0, cat1, cat2, nrm0, nrm1, nrm2):

    @pl.when(pl.program_id(0) == 0)
    def _prep():
        we0[...] = eW0[...].astype(jnp.bfloat16)
        we1[...] = eW1[...].astype(jnp.bfloat16)
        we2[...] = eW2[...].astype(jnp.bfloat16)
        wd0[...] = dW0[...].astype(jnp.bfloat16)
        wd1[...] = dW1[...].astype(jnp.bfloat16)
        wd2[...] = dW2[...].astype(jnp.bfloat16)
        for cb_ref, hi_ref, cat_ref, nrm_ref in (
                (cb0, hi0, cat0, nrm0), (cb1, hi1, cat1, nrm1),
                (cb2, hi2, cat2, nrm2)):
            cb = cb_ref[...]
            hi, cat = _split_cat(cb)
            hi_ref[...] = hi
            cat_ref[...] = cat
            nrm_ref[...] = jax.lax.dot_general(
                jnp.ones((1, cb.shape[1]), jnp.float32), cb * cb,
                dimension_numbers=_MMT, preferred_element_type=jnp.float32,
                precision=jax.lax.Precision.HIGHEST)

    x = x_ref[...]
    xb = x.astype(jnp.bfloat16)
    h = jnp.maximum(_bdot(xb, we0[...]) + eb0[...], 0.0)
    h = jnp.maximum(_bdot(h.astype(jnp.bfloat16), we1[...]) + eb1[...], 0.0)
    z = _bdot(h.astype(jnp.bfloat16), we2[...]) + eb2[...]

    mu = jnp.mean(z, axis=-1, keepdims=True)
    zc = z - mu
    var = jnp.mean(zc * zc, axis=-1, keepdims=True)
    z = zc / jnp.sqrt(var + 1e-5) * ln_g[...] + ln_b[...]

    zq0, l0, i0 = _vq_stage(z, hi0[...], cat0[...], nrm0[...])
    r1 = z - zq0
    zq1, l1, i1 = _vq_stage(r1, hi1[...], cat1[...], nrm1[...])
    r2 = r1 - zq1
    zq2, l2, i2 = _vq_stage(r2, hi2[...], cat2[...], nrm2[...])

    zqt = zq0 + zq1 + zq2
    h = jnp.maximum(_bdot(zqt.astype(jnp.bfloat16), wd0[...]) + db0[...], 0.0)
    h = jnp.maximum(_bdot(h.astype(jnp.bfloat16), wd1[...]) + db1[...], 0.0)
    xh = _bdot(h.astype(jnp.bfloat16), wd2[...]) + db2[...]

    dx = xh - x
    recon = jnp.mean(dx * dx, axis=1)
    loss = l0 + l1 + l2 + recon

    xhat_ref[...] = xh
    loss_ref[...] = loss[:, None]
    idx_ref[...] = jnp.concatenate(
        [i0[:, None], i1[:, None], i2[:, None]], axis=1)


def kernel(x, enc_W0, enc_b0, enc_W1, enc_b1, enc_W2, enc_b2,
           dec_W0, dec_b0, dec_W1, dec_b1, dec_W2, dec_b2,
           ln_g, ln_b, cb0, cb1, cb2):
    batch, in_dim = x.shape
    k, lat = cb0.shape
    grid = (batch // _BT,)

    def rows(i):
        return (i, 0)

    def whole(i):
        return (0, 0)

    row_spec = lambda n: pl.BlockSpec((_BT, n), rows)
    full_spec = lambda a: pl.BlockSpec(a.shape, whole)

    b2 = lambda v: v.reshape(1, -1)
    ins = (x,
           enc_W0, b2(enc_b0), enc_W1, b2(enc_b1), enc_W2, b2(enc_b2),
           dec_W0, b2(dec_b0), dec_W1, b2(dec_b1), dec_W2, b2(dec_b2),
           b2(ln_g), b2(ln_b), cb0, cb1, cb2)
    in_specs = [row_spec(in_dim)] + [full_spec(a) for a in ins[1:]]

    out_shape = (
        jax.ShapeDtypeStruct((batch, in_dim), jnp.float32),
        jax.ShapeDtypeStruct((batch, 1), jnp.float32),
        jax.ShapeDtypeStruct((batch, 3), jnp.int32),
    )
    out_specs = (
        row_spec(in_dim),
        pl.BlockSpec((_BT, 1), rows),
        pl.BlockSpec((_BT, 3), rows),
    )

    bf = jnp.bfloat16
    scratch = [
        pltpu.VMEM(enc_W0.shape, bf), pltpu.VMEM(enc_W1.shape, bf),
        pltpu.VMEM(enc_W2.shape, bf), pltpu.VMEM(dec_W0.shape, bf),
        pltpu.VMEM(dec_W1.shape, bf), pltpu.VMEM(dec_W2.shape, bf),
        pltpu.VMEM((k, lat), bf), pltpu.VMEM((k, lat), bf),
        pltpu.VMEM((k, lat), bf),
        pltpu.VMEM((k, 3 * lat), bf), pltpu.VMEM((k, 3 * lat), bf),
        pltpu.VMEM((k, 3 * lat), bf),
        pltpu.VMEM((1, k), jnp.float32), pltpu.VMEM((1, k), jnp.float32),
        pltpu.VMEM((1, k), jnp.float32),
    ]

    xh, loss, indices = pl.pallas_call(
        _body,
        grid=grid,
        in_specs=in_specs,
        out_specs=out_specs,
        out_shape=out_shape,
        scratch_shapes=scratch,
    )(*ins)

    return xh, loss.reshape(batch), indices
